# baseline (speedup 1.0000x reference)
import functools

import jax
import jax.numpy as jnp
from jax import lax
from jax.experimental import pallas as pl
from jax.experimental.pallas import tpu as pltpu
from jax.experimental.pallas import tpu_sc as plsc

L = 16
B = 4
N = 4096
NWB = 8
RS = N // NWB
RC = RS // L
CSTRIP = N // NWB
CM = N * L
INF = 3.0e38

_MESH = plsc.VectorSubcoreMesh(core_axis_name="c", subcore_axis_name="s")


@functools.partial(
    pl.kernel,
    out_type=(
        jax.ShapeDtypeStruct((2 * L * NWB * B,), jnp.float32),
        jax.ShapeDtypeStruct((2 * 16 * CM,), jnp.float32),
    ),
    mesh=_MESH,
    scratch_types=[
        pltpu.VMEM((3 * N,), jnp.float32),
        pltpu.VMEM((N,), jnp.float32),
        pltpu.VMEM((3 * RS,), jnp.float32),
        pltpu.VMEM((RS,), jnp.float32),
        pltpu.VMEM((CM,), jnp.float32),
        pltpu.VMEM((CSTRIP * L,), jnp.float32),
        pltpu.VMEM((CSTRIP * L,), jnp.float32),
        pltpu.VMEM((2 * L,), jnp.float32),
    ],
)
def _cd_kernel(p1_hbm, p2_hbm, out_hbm, stage_hbm, p2v, hv, p1v, n1v, cmv,
               accv, tmpv, obuf):
    cid = lax.axis_index("c")
    sid = lax.axis_index("s")
    b = cid * 2 + sid // NWB
    k = sid % NWB
    rbase = k * RS

    pltpu.sync_copy(p2_hbm.at[pl.ds(b * 3 * N, 3 * N)], p2v)
    for d in range(3):
        pltpu.sync_copy(
            p1_hbm.at[pl.ds(b * 3 * N + d * N + rbase, RS)],
            p1v.at[pl.ds(d * RS, RS)])

    def _prep_p2(i, carry):
        s = pl.ds(i * L, L)
        x = p2v[pl.ds(i * L, L)]
        y = p2v[pl.ds(N + i * L, L)]
        z = p2v[pl.ds(2 * N + i * L, L)]
        hv[s] = x * x + y * y + z * z
        p2v[pl.ds(i * L, L)] = x * -2.0
        p2v[pl.ds(N + i * L, L)] = y * -2.0
        p2v[pl.ds(2 * N + i * L, L)] = z * -2.0
        return carry

    lax.fori_loop(0, N // L, _prep_p2, 0)

    def _prep_p1(i, carry):
        s = pl.ds(i * L, L)
        x = p1v[pl.ds(i * L, L)]
        y = p1v[pl.ds(RS + i * L, L)]
        z = p1v[pl.ds(2 * RS + i * L, L)]
        n1v[s] = x * x + y * y + z * z
        return carry

    lax.fori_loop(0, RS // L, _prep_p1, 0)

    def _init_cm(i, carry):
        cmv[pl.ds(i * L, L)] = jnp.full((L,), INF, jnp.float32)
        return carry

    lax.fori_loop(0, N, _init_cm, 0)

    def _row_chunk(rc, d1acc):
        x1 = p1v[pl.ds(rc * L, L)]
        y1 = p1v[pl.ds(RS + rc * L, L)]
        z1 = p1v[pl.ds(2 * RS + rc * L, L)]
        n1 = n1v[pl.ds(rc * L, L)]

        def _cchunk(jc, rm):
            cb = jc * L
            hc = hv[pl.ds(cb, L)]
            ac = p2v[pl.ds(cb, L)]
            bc = p2v[pl.ds(N + cb, L)]
            cc = p2v[pl.ds(2 * N + cb, L)]
            for l in range(L):
                t = (n1 + hc[l]) + x1 * ac[l] + y1 * bc[l] + z1 * cc[l]
                cs = pl.ds((cb + l) * L, L)
                cmv[cs] = jnp.minimum(cmv[cs], t)
                rm = jnp.minimum(rm, t)
            return rm

        rm = lax.fori_loop(0, N // L, _cchunk,
                           jnp.full((L,), INF, jnp.float32))
        return d1acc + rm

    d1vec = lax.fori_loop(0, RC, _row_chunk, jnp.zeros((L,), jnp.float32))

    gwid = cid * 16 + sid
    pltpu.sync_copy(cmv, stage_hbm.at[pl.ds(gwid * CM, CM)])
    plsc.subcore_barrier()

    gbase = cid * 16 + (sid // NWB) * NWB
    fbase = k * CSTRIP * L
    pltpu.sync_copy(stage_hbm.at[pl.ds(gbase * CM + fbase, CSTRIP * L)],
                    accv)
    for srel in range(1, NWB):
        pltpu.sync_copy(
            stage_hbm.at[pl.ds((gbase + srel) * CM + fbase, CSTRIP * L)],
            tmpv)

        def _minacc(i, carry):
            s = pl.ds(i * L, L)
            accv[s] = jnp.minimum(accv[s], tmpv[s])
            return carry

        lax.fori_loop(0, CSTRIP * L // L, _minacc, 0)

    def _colred(j, d2sum):
        v = accv[pl.ds(j * L, L)]
        m = v[0]
        for l in range(1, L):
            m = jnp.minimum(m, v[l])
        return d2sum + m

    d2sum = lax.fori_loop(0, CSTRIP, _colred, jnp.float32(0.0))

    iota = lax.iota(jnp.int32, L)
    obuf[pl.ds(0, L)] = d1vec
    obuf[pl.ds(L, L)] = jnp.where(iota == 0, d2sum, jnp.float32(0.0))
    pltpu.sync_copy(obuf, out_hbm.at[pl.ds(gwid * 2 * L, 2 * L)])


def kernel(p1, p2):
    p1t = jnp.transpose(p1, (0, 2, 1)).reshape(B * 3 * N)
    p2t = jnp.transpose(p2, (0, 2, 1)).reshape(B * 3 * N)
    out, _ = _cd_kernel(p1t, p2t)
    return jnp.sum(out) * (1.0 / (B * N))
